# baseline (speedup 1.0000x reference)
import functools

import jax
import jax.numpy as jnp
from jax import lax
from jax.experimental import pallas as pl
from jax.experimental.pallas import tpu as pltpu
from jax.experimental.pallas import tpu_sc as plsc

NC = 2
NS = 16
NW = NC * NS
CH = 128
NCH = 13
B_PER_W = NCH * CH
B_PAD = NW * B_PER_W
D = 64
DP = 2 * D
EPS = 1e-5


def _sc_gather_stats(n_total, x_hbm, table_hbm, h_hbm, stats_hbm,
                     idx_v, buf_v, acc_v, gsem0, gsem1, wsem0, wsem1):
    gsems = (gsem0, gsem1)
    wsems = (wsem0, wsem1)
    wid = lax.axis_index("s") * NC + lax.axis_index("c")
    base = wid * B_PER_W
    n_real = jnp.maximum(0, jnp.minimum(B_PER_W, n_total - base))

    pltpu.sync_copy(x_hbm.at[wid], idx_v)

    def gather(j):
        return pltpu.async_copy(table_hbm.at[idx_v.at[j]],
                                buf_v.at[j % 2], gsems[j % 2])

    carry = (jnp.zeros((16,), jnp.float32),) * 8
    gdesc = gather(0)
    wdescs = [None, None]
    for j in range(NCH):
        b = j % 2
        if j + 1 < NCH:
            if wdescs[1 - b] is not None:
                wdescs[1 - b].wait()
            next_gdesc = gather(j + 1)
        gdesc.wait()

        nj = jnp.maximum(0, jnp.minimum(CH, n_real - j * CH))

        def row_body(i, c, _b=b):
            accs = list(c)
            for k in range(4):
                v = buf_v[_b, i, pl.ds(16 * k, 16)]
                accs[k] = accs[k] + v
                accs[4 + k] = accs[4 + k] + v * v
            return tuple(accs)

        carry = lax.fori_loop(0, nj, row_body, carry)
        wdescs[b] = pltpu.async_copy(
            buf_v.at[b], h_hbm.at[pl.ds(base + j * CH, CH)], wsems[b])
        if j + 1 < NCH:
            gdesc = next_gdesc

    for k in range(4):
        acc_v[0, pl.ds(16 * k, 16)] = carry[k]
        acc_v[1, pl.ds(16 * k, 16)] = carry[4 + k]
    pltpu.sync_copy(acc_v.at[0], stats_hbm.at[0, wid])
    pltpu.sync_copy(acc_v.at[1], stats_hbm.at[1, wid])

    wdescs[0].wait()
    wdescs[1].wait()


def _tc_affine(n_total, stats_ref, w_ref, b_ref, ms_ref, h_ref, o_ref):
    inv_n = 1.0 / n_total
    m1 = jnp.sum(stats_ref[0], axis=0) * inv_n
    q = jnp.sum(stats_ref[1], axis=0) * inv_n
    w = w_ref[0]
    b = b_ref[0]
    ms = ms_ref[0]
    v1 = q - ms * m1 * m1 * (2.0 - ms)
    r1 = lax.rsqrt(v1 + EPS)
    a1 = w * r1
    c1 = b - a1 * m1 * ms
    m2 = a1 * m1 + c1
    c2 = c1 - m2 * ms
    v2 = a1 * a1 * q + 2.0 * a1 * c2 * m1 + c2 * c2
    r2 = lax.rsqrt(v2 + EPS)
    a_f = w * r2 * a1
    c_f = w * r2 * c2 + b
    a2 = jnp.concatenate([a_f, a_f])
    c2w = jnp.concatenate([c_f, c_f])
    o_ref[...] = h_ref[...] * a2[None, :] + c2w[None, :]


def kernel(x, edge_index, edge_weight, subG_nodes, batch_nodes,
           batch_nodes_mask, emb_table, gn_weight, gn_bias, gn_mean_scale):
    n_total = x.shape[0]
    xi = x.astype(jnp.int32)
    x_pad = jnp.pad(xi, (0, B_PAD - n_total)).reshape(NW, NCH, CH)
    table128 = jnp.pad(emb_table, ((0, 0), (0, DP - D)))

    mesh = plsc.VectorSubcoreMesh(core_axis_name="c", subcore_axis_name="s")
    sc_fn = pl.kernel(
        functools.partial(_sc_gather_stats, n_total),
        out_type=[
            jax.ShapeDtypeStruct((B_PAD, DP), jnp.float32),
            jax.ShapeDtypeStruct((2, NW, D), jnp.float32),
        ],
        mesh=mesh,
        scratch_types=[
            pltpu.VMEM((NCH, CH), jnp.int32),
            pltpu.VMEM((2, CH, DP), jnp.float32),
            pltpu.VMEM((2, D), jnp.float32),
            pltpu.SemaphoreType.DMA,
            pltpu.SemaphoreType.DMA,
            pltpu.SemaphoreType.DMA,
            pltpu.SemaphoreType.DMA,
        ],
    )
    h, stats = sc_fn(x_pad, table128)

    grid = 8
    rb = B_PAD // grid
    out_pad = pl.pallas_call(
        functools.partial(_tc_affine, n_total),
        grid=(grid,),
        in_specs=[
            pl.BlockSpec((2, NW, D), lambda i: (0, 0, 0)),
            pl.BlockSpec((1, D), lambda i: (0, 0)),
            pl.BlockSpec((1, D), lambda i: (0, 0)),
            pl.BlockSpec((1, D), lambda i: (0, 0)),
            pl.BlockSpec((rb, DP), lambda i: (i, 0)),
        ],
        out_specs=pl.BlockSpec((rb, DP), lambda i: (i, 0)),
        out_shape=jax.ShapeDtypeStruct((B_PAD, DP), jnp.float32),
    )(stats, gn_weight.reshape(1, D), gn_bias.reshape(1, D),
      gn_mean_scale.reshape(1, D), h)

    return out_pad[:n_total, :D]
